# Initial kernel scaffold; baseline (speedup 1.0000x reference)
import functools

import jax
import jax.numpy as jnp
from jax import lax
from jax.experimental import pallas as pl
from jax.experimental.pallas import tpu as pltpu
from jax.experimental.pallas import tpu_sc as plsc

NUM_EMB = 1000000
DIM = 64
BATCH = 4096
SEQ = 50
TOTAL = BATCH * SEQ
NW = 32
PER_W = TOTAL // NW
GRP = 128
GROUPS_PER_W = PER_W // GRP
G_PER_CHUNK = 5
CHUNK = G_PER_CHUNK * GRP
NCHUNK = GROUPS_PER_W // G_PER_CHUNK

_mesh = plsc.VectorSubcoreMesh(core_axis_name="c", subcore_axis_name="s")


@functools.partial(
    pl.kernel,
    out_type=jax.ShapeDtypeStruct((TOTAL, 2 * DIM), jnp.float32),
    mesh=_mesh,
    scratch_types=[
        pltpu.VMEM((GROUPS_PER_W, GRP), jnp.int32),
        pltpu.VMEM((CHUNK, DIM), jnp.float32),
        pltpu.VMEM((CHUNK, DIM), jnp.float32),
        pltpu.SemaphoreType.DMA,
        pltpu.SemaphoreType.DMA,
    ],
)
def _emb_lookup(x_hbm, g_hbm, r_hbm, out_hbm, idx_v, gbuf, rbuf, sem_g, sem_r):
    wid = lax.axis_index("s") * 2 + lax.axis_index("c")
    pltpu.sync_copy(x_hbm.at[pl.ds(wid * GROUPS_PER_W, GROUPS_PER_W)], idx_v)

    def body(i, carry):
        copies = []
        for j in range(G_PER_CHUNK):
            g = i * G_PER_CHUNK + j
            dst = pl.ds(j * GRP, GRP)
            copies.append(
                pltpu.async_copy(g_hbm.at[idx_v.at[g]], gbuf.at[dst], sem_g))
            copies.append(
                pltpu.async_copy(r_hbm.at[idx_v.at[g]], rbuf.at[dst], sem_r))
        for c in copies:
            c.wait()
        base = wid * PER_W + i * CHUNK
        pltpu.sync_copy(gbuf, out_hbm.at[pl.ds(base, CHUNK), pl.ds(0, DIM)])
        pltpu.sync_copy(rbuf, out_hbm.at[pl.ds(base, CHUNK), pl.ds(DIM, DIM)])
        return carry

    lax.fori_loop(0, NCHUNK, body, 0)


def kernel(x, glove_weight, rand_weight):
    x2 = x.reshape(TOTAL // GRP, GRP).astype(jnp.int32)
    out = _emb_lookup(x2, glove_weight, rand_weight)
    return out.reshape(BATCH, SEQ, 2 * DIM)


# speedup vs baseline: 1.2428x; 1.2428x over previous
import functools

import jax
import jax.numpy as jnp
from jax import lax
from jax.experimental import pallas as pl
from jax.experimental.pallas import tpu as pltpu
from jax.experimental.pallas import tpu_sc as plsc

NUM_EMB = 1000000
DIM = 64
BATCH = 4096
SEQ = 50
TOTAL = BATCH * SEQ
NW = 32
PER_W = TOTAL // NW
GRP = 128
GROUPS_PER_W = PER_W // GRP
G_PER_CHUNK = 5
CHUNK = G_PER_CHUNK * GRP
NCHUNK = GROUPS_PER_W // G_PER_CHUNK

_mesh = plsc.VectorSubcoreMesh(core_axis_name="c", subcore_axis_name="s")


@functools.partial(
    pl.kernel,
    out_type=jax.ShapeDtypeStruct((TOTAL, 2 * DIM), jnp.float32),
    mesh=_mesh,
    compiler_params=pltpu.CompilerParams(use_tc_tiling_on_sc=False),
    scratch_types=[
        pltpu.VMEM((PER_W,), jnp.int32),
        pltpu.VMEM((CHUNK, DIM), jnp.float32),
        pltpu.VMEM((CHUNK, DIM), jnp.float32),
        pltpu.SemaphoreType.DMA,
        pltpu.SemaphoreType.DMA,
    ],
)
def _emb_lookup(x_hbm, g_hbm, r_hbm, out_hbm, idx_v, gbuf, rbuf, sem_g, sem_r):
    wid = lax.axis_index("s") * 2 + lax.axis_index("c")
    pltpu.sync_copy(x_hbm.at[pl.ds(wid * PER_W, PER_W)], idx_v)

    def body(i, carry):
        copies = []
        for j in range(G_PER_CHUNK):
            src = idx_v.at[pl.ds((i * G_PER_CHUNK + j) * GRP, GRP)]
            dst = pl.ds(j * GRP, GRP)
            copies.append(
                pltpu.async_copy(g_hbm.at[src], gbuf.at[dst], sem_g))
            copies.append(
                pltpu.async_copy(r_hbm.at[src], rbuf.at[dst], sem_r))
        for c in copies:
            c.wait()
        base = wid * PER_W + i * CHUNK
        pltpu.sync_copy(gbuf, out_hbm.at[pl.ds(base, CHUNK), pl.ds(0, DIM)])
        pltpu.sync_copy(rbuf, out_hbm.at[pl.ds(base, CHUNK), pl.ds(DIM, DIM)])
        return carry

    lax.fori_loop(0, NCHUNK, body, 0)


def kernel(x, glove_weight, rand_weight):
    x2 = x.reshape(TOTAL).astype(jnp.int32)
    out = _emb_lookup(x2, glove_weight, rand_weight)
    return out.reshape(BATCH, SEQ, 2 * DIM)
